# baseline (speedup 1.0000x reference)
import jax
import jax.numpy as jnp
from jax.experimental import pallas as pl
from jax.experimental.pallas import tpu as pltpu

_BM = 256


def _mm_kernel(x_ref, w_ref, b_ref, o_ref, w_bf_ref):
    i = pl.program_id(0)

    @pl.when(i == 0)
    def _():
        w_bf_ref[...] = w_ref[...].astype(jnp.bfloat16)

    acc = jnp.dot(
        x_ref[...].astype(jnp.bfloat16),
        w_bf_ref[...],
        preferred_element_type=jnp.float32,
    )
    o_ref[...] = acc + b_ref[...]


def kernel(x, W_router, b_router, W0, b0):
    B, S, D = x.shape
    M = B * S
    x2 = x.reshape(M, D)
    b2 = b0.reshape(1, D)

    out = pl.pallas_call(
        _mm_kernel,
        grid=(M // _BM,),
        in_specs=[
            pl.BlockSpec((_BM, D), lambda i: (i, 0)),
            pl.BlockSpec((D, D), lambda i: (0, 0)),
            pl.BlockSpec((1, D), lambda i: (0, 0)),
        ],
        out_specs=pl.BlockSpec((_BM, D), lambda i: (i, 0)),
        out_shape=jax.ShapeDtypeStruct((M, D), jnp.float32),
        scratch_shapes=[pltpu.VMEM((D, D), jnp.bfloat16)],
        compiler_params=pltpu.CompilerParams(
            dimension_semantics=("arbitrary",),
            vmem_limit_bytes=100 * 1024 * 1024,
        ),
    )(x2, W0, b2)
    return out.reshape(B, S, D)


# speedup vs baseline: 1.0682x; 1.0682x over previous
import jax
import jax.numpy as jnp
from jax.experimental import pallas as pl
from jax.experimental.pallas import tpu as pltpu

_BM = 512


def _mm_kernel(x_ref, w_ref, b_ref, o_ref, w_bf_ref):
    i = pl.program_id(0)

    @pl.when(i == 0)
    def _():
        w_bf_ref[...] = w_ref[...].astype(jnp.bfloat16)

    acc = jnp.dot(
        x_ref[...].astype(jnp.bfloat16),
        w_bf_ref[...],
        preferred_element_type=jnp.float32,
    )
    o_ref[...] = acc + b_ref[...]


def kernel(x, W_router, b_router, W0, b0):
    B, S, D = x.shape
    M = B * S
    x2 = x.reshape(M, D)
    b2 = b0.reshape(1, D)

    out = pl.pallas_call(
        _mm_kernel,
        grid=(M // _BM,),
        in_specs=[
            pl.BlockSpec((_BM, D), lambda i: (i, 0)),
            pl.BlockSpec((D, D), lambda i: (0, 0)),
            pl.BlockSpec((1, D), lambda i: (0, 0)),
        ],
        out_specs=pl.BlockSpec((_BM, D), lambda i: (i, 0)),
        out_shape=jax.ShapeDtypeStruct((M, D), jnp.float32),
        scratch_shapes=[pltpu.VMEM((D, D), jnp.bfloat16)],
        compiler_params=pltpu.CompilerParams(
            dimension_semantics=("arbitrary",),
        ),
    )(x2, W0, b2)
    return out.reshape(B, S, D)
